# initial kernel scaffold (unmeasured)
import jax
import jax.numpy as jnp
from jax import lax
from jax.experimental import pallas as pl
from jax.experimental.pallas import tpu as pltpu

N_DEV = 16
SQ = 2048
D = 1024
HQ = 8
DH = 128
N_PHASE = 4
PH = SQ // N_PHASE
CHUNK = SQ // N_DEV
SCALE = 0.08838834764831843


def kernel(x, Wq, K_ext, V_ext, Wo):
    xb = x[0].astype(jnp.bfloat16)
    wqb = Wq.astype(jnp.bfloat16)
    kb = K_ext[0].reshape(SQ, D).astype(jnp.bfloat16)
    vb = V_ext[0].reshape(SQ, D).astype(jnp.bfloat16)
    wob = Wo.astype(jnp.bfloat16)

    def body(x_ref, wq_ref, k_ref, v_ref, wo_ref, out_ref,
             o_ref, l_ref, ctx_ref, ob_ref, lb_ref,
             rs_send_sem, rs_recv_sems, rsl_send_sem, rsl_recv_sems,
             ag_send_sem, ag_recv_sems):
        me = lax.axis_index("i")
        left = (me - 1) % N_DEV
        right = (me + 1) % N_DEV

        barrier_sem = pltpu.get_barrier_semaphore()
        for nbr in (left, right):
            pl.semaphore_signal(barrier_sem, inc=1, device_id=(nbr,),
                                device_id_type=pl.DeviceIdType.MESH)
        pl.semaphore_wait(barrier_sem, 2)

        q = jnp.dot(x_ref[...], wq_ref[...],
                    preferred_element_type=jnp.float32)
        q16 = q.astype(jnp.bfloat16)

        k = k_ref[...]
        v = v_ref[...]
        for c in range(N_PHASE):
            qc = q16.reshape(8, 4, 64, D)[:, c].reshape(PH, D)
            kc = k.reshape(8, 4, 64, D)[:, c].reshape(PH, D)
            vc = v.reshape(8, 4, 64, D)[:, c].reshape(PH, D)
            for h in range(HQ):
                s = lax.dot_general(
                    qc[:, h * DH:(h + 1) * DH], kc[:, h * DH:(h + 1) * DH],
                    (((1,), (1,)), ((), ())),
                    preferred_element_type=jnp.float32)
                w = jnp.exp(s * SCALE)
                lsum = jnp.sum(w, axis=1)
                o = lax.dot_general(
                    w.astype(jnp.bfloat16), vc[:, h * DH:(h + 1) * DH],
                    (((1,), (0,)), ((), ())),
                    preferred_element_type=jnp.float32)
                o_ref[pl.ds(PH * c, PH), pl.ds(h * DH, DH)] = o
                l_ref[pl.ds(h, 1), pl.ds(PH * c, PH)] = lsum[None, :]

        for t in range(N_DEV - 1):
            sc = (me - 1 - t) % N_DEV
            rc = (me - 2 - t) % N_DEV
            rdma_o = pltpu.make_async_remote_copy(
                src_ref=o_ref.at[pl.ds(sc * CHUNK, CHUNK), :],
                dst_ref=ob_ref.at[t],
                send_sem=rs_send_sem,
                recv_sem=rs_recv_sems.at[t],
                device_id=(right,),
                device_id_type=pl.DeviceIdType.MESH)
            rdma_l = pltpu.make_async_remote_copy(
                src_ref=l_ref.at[:, pl.ds(sc * CHUNK, CHUNK)],
                dst_ref=lb_ref.at[t],
                send_sem=rsl_send_sem,
                recv_sem=rsl_recv_sems.at[t],
                device_id=(right,),
                device_id_type=pl.DeviceIdType.MESH)
            rdma_o.start()
            rdma_l.start()
            rdma_o.wait()
            rdma_l.wait()
            o_ref[pl.ds(rc * CHUNK, CHUNK), :] = (
                ob_ref[t] + o_ref[pl.ds(rc * CHUNK, CHUNK), :])
            l_ref[:, pl.ds(rc * CHUNK, CHUNK)] = (
                lb_ref[t] + l_ref[:, pl.ds(rc * CHUNK, CHUNK)])

        oc = o_ref[pl.ds(me * CHUNK, CHUNK), :]
        lc = l_ref[:, pl.ds(me * CHUNK, CHUNK)]
        ctx = (oc.reshape(CHUNK, HQ, DH) / lc.T[:, :, None]).reshape(CHUNK, D)
        ctx_ref[pl.ds(me * CHUNK, CHUNK), :] = ctx.astype(jnp.bfloat16)

        for t in range(N_DEV - 1):
            sc = (me - t) % N_DEV
            rdma = pltpu.make_async_remote_copy(
                src_ref=ctx_ref.at[pl.ds(sc * CHUNK, CHUNK), :],
                dst_ref=ctx_ref.at[pl.ds(sc * CHUNK, CHUNK), :],
                send_sem=ag_send_sem,
                recv_sem=ag_recv_sems.at[t],
                device_id=(right,),
                device_id_type=pl.DeviceIdType.MESH)
            rdma.start()
            rdma.wait()

        ctx_all = ctx_ref[...]
        ctx_nat = (ctx_all.reshape(N_PHASE, 8, 64, D)
                   .transpose(1, 0, 2, 3).reshape(SQ, D))
        out_ref[...] = jnp.dot(ctx_nat, wo_ref[...],
                               preferred_element_type=jnp.float32)

    out = pl.pallas_call(
        body,
        out_shape=jax.ShapeDtypeStruct((SQ, D), jnp.float32),
        in_specs=[pl.BlockSpec(memory_space=pltpu.VMEM)] * 5,
        out_specs=pl.BlockSpec(memory_space=pltpu.VMEM),
        scratch_shapes=[
            pltpu.VMEM((SQ, D), jnp.float32),
            pltpu.VMEM((HQ, SQ), jnp.float32),
            pltpu.VMEM((SQ, D), jnp.bfloat16),
            pltpu.VMEM((N_DEV - 1, CHUNK, D), jnp.float32),
            pltpu.VMEM((N_DEV - 1, HQ, CHUNK), jnp.float32),
            pltpu.SemaphoreType.DMA,
            pltpu.SemaphoreType.DMA((N_DEV - 1,)),
            pltpu.SemaphoreType.DMA,
            pltpu.SemaphoreType.DMA((N_DEV - 1,)),
            pltpu.SemaphoreType.DMA,
            pltpu.SemaphoreType.DMA((N_DEV - 1,)),
        ],
        compiler_params=pltpu.CompilerParams(collective_id=0),
    )(xb, wqb, kb, vb, wob)
    return out[None]


# baseline (device time: 250073 ns/iter reference)
import jax
import jax.numpy as jnp
from jax import lax
from jax.experimental import pallas as pl
from jax.experimental.pallas import tpu as pltpu

N_DEV = 16
SQ = 2048
D = 1024
HQ = 8
DH = 128
N_PHASE = 4
PH = SQ // N_PHASE
CHUNK = SQ // N_DEV
SCALE = 0.08838834764831843


def kernel(x, Wq, K_ext, V_ext, Wo):
    xb = x[0].astype(jnp.bfloat16)
    wqb = Wq.astype(jnp.bfloat16)
    kb = K_ext[0].reshape(SQ, D).astype(jnp.bfloat16)
    vb = V_ext[0].reshape(SQ, D).astype(jnp.bfloat16)
    wob = Wo.astype(jnp.bfloat16)

    def body(x_ref, wq_ref, k_ref, v_ref, wo_ref, out_ref,
             o_ref, l_ref, ctx_ref, ob_ref, lb_ref,
             rs_send_sem, rs_recv_sems, rsl_send_sem, rsl_recv_sems,
             ag_send_sem, ag_recv_sems):
        me = lax.axis_index("i")
        left = (me - 1) % N_DEV
        right = (me + 1) % N_DEV

        barrier_sem = pltpu.get_barrier_semaphore()
        for nbr in (left, right):
            pl.semaphore_signal(barrier_sem, inc=1, device_id=(nbr,),
                                device_id_type=pl.DeviceIdType.MESH)
        pl.semaphore_wait(barrier_sem, 2)

        q = jnp.dot(x_ref[...], wq_ref[...],
                    preferred_element_type=jnp.float32)
        q16 = q.astype(jnp.bfloat16)

        k = k_ref[...]
        v = v_ref[...]
        for c in range(N_PHASE):
            qc = q16.reshape(8, 4, 64, D)[:, c].reshape(PH, D)
            kc = k.reshape(8, 4, 64, D)[:, c].reshape(PH, D)
            vc = v.reshape(8, 4, 64, D)[:, c].reshape(PH, D)
            for h in range(HQ):
                s = lax.dot_general(
                    qc[:, h * DH:(h + 1) * DH], kc[:, h * DH:(h + 1) * DH],
                    (((1,), (1,)), ((), ())),
                    preferred_element_type=jnp.float32)
                w = jnp.exp(s * SCALE)
                lsum = jnp.sum(w, axis=1)
                o = lax.dot_general(
                    w.astype(jnp.bfloat16), vc[:, h * DH:(h + 1) * DH],
                    (((1,), (0,)), ((), ())),
                    preferred_element_type=jnp.float32)
                o_ref[pl.ds(PH * c, PH), pl.ds(h * DH, DH)] = o
                l_ref[pl.ds(h, 1), pl.ds(PH * c, PH)] = lsum[None, :]

        for t in range(N_DEV - 1):
            sc = (me - 1 - t) % N_DEV
            rc = (me - 2 - t) % N_DEV
            rdma_o = pltpu.make_async_remote_copy(
                src_ref=o_ref.at[pl.ds(sc * CHUNK, CHUNK), :],
                dst_ref=ob_ref.at[t],
                send_sem=rs_send_sem,
                recv_sem=rs_recv_sems.at[t],
                device_id=(right,),
                device_id_type=pl.DeviceIdType.MESH)
            rdma_l = pltpu.make_async_remote_copy(
                src_ref=l_ref.at[:, pl.ds(sc * CHUNK, CHUNK)],
                dst_ref=lb_ref.at[t],
                send_sem=rsl_send_sem,
                recv_sem=rsl_recv_sems.at[t],
                device_id=(right,),
                device_id_type=pl.DeviceIdType.MESH)
            rdma_o.start()
            rdma_l.start()
            rdma_o.wait()
            rdma_l.wait()
            o_ref[pl.ds(rc * CHUNK, CHUNK), :] = (
                ob_ref[t] + o_ref[pl.ds(rc * CHUNK, CHUNK), :])
            l_ref[:, pl.ds(rc * CHUNK, CHUNK)] = (
                lb_ref[t] + l_ref[:, pl.ds(rc * CHUNK, CHUNK)])

        oc = o_ref[pl.ds(me * CHUNK, CHUNK), :]
        lc = l_ref[:, pl.ds(me * CHUNK, CHUNK)]
        ctx = (oc.reshape(CHUNK, HQ, DH) / lc.T[:, :, None]).reshape(CHUNK, D)
        ctx_ref[pl.ds(me * CHUNK, CHUNK), :] = ctx.astype(jnp.bfloat16)

        for t in range(N_DEV - 1):
            sc = (me - t) % N_DEV
            rdma = pltpu.make_async_remote_copy(
                src_ref=ctx_ref.at[pl.ds(sc * CHUNK, CHUNK), :],
                dst_ref=ctx_ref.at[pl.ds(sc * CHUNK, CHUNK), :],
                send_sem=ag_send_sem,
                recv_sem=ag_recv_sems.at[t],
                device_id=(right,),
                device_id_type=pl.DeviceIdType.MESH)
            rdma.start()
            rdma.wait()

        ctx_all = ctx_ref[...]
        ctx_nat = (ctx_all.reshape(N_PHASE, 8, 64, D)
                   .transpose(1, 0, 2, 3).reshape(SQ, D))
        out_ref[...] = jnp.dot(ctx_nat, wo_ref[...],
                               preferred_element_type=jnp.float32)

    out = pl.pallas_call(
        body,
        out_shape=jax.ShapeDtypeStruct((SQ, D), jnp.float32),
        in_specs=[pl.BlockSpec(memory_space=pltpu.VMEM)] * 5,
        out_specs=pl.BlockSpec(memory_space=pltpu.VMEM),
        scratch_shapes=[
            pltpu.VMEM((SQ, D), jnp.float32),
            pltpu.VMEM((HQ, SQ), jnp.float32),
            pltpu.VMEM((SQ, D), jnp.bfloat16),
            pltpu.VMEM((N_DEV - 1, CHUNK, D), jnp.float32),
            pltpu.VMEM((N_DEV - 1, HQ, CHUNK), jnp.float32),
            pltpu.SemaphoreType.DMA,
            pltpu.SemaphoreType.DMA((N_DEV - 1,)),
            pltpu.SemaphoreType.DMA,
            pltpu.SemaphoreType.DMA((N_DEV - 1,)),
            pltpu.SemaphoreType.DMA,
            pltpu.SemaphoreType.DMA((N_DEV - 1,)),
        ],
        compiler_params=pltpu.CompilerParams(
            collective_id=0, vmem_limit_bytes=100 * 1024 * 1024),
    )(xb, wqb, kb, vb, wob)
    return out[None]


# device time: 226999 ns/iter; 1.1016x vs baseline; 1.1016x over previous
import jax
import jax.numpy as jnp
from jax import lax
from jax.experimental import pallas as pl
from jax.experimental.pallas import tpu as pltpu

N_DEV = 16
SQ = 2048
D = 1024
HQ = 8
DH = 128
N_PHASE = 4
PH = SQ // N_PHASE
CHUNK = SQ // N_DEV
HALF = D // 2
LH = HQ // 2
SCALE = 0.08838834764831843


def kernel(x, Wq, K_ext, V_ext, Wo):
    xb = x[0].astype(jnp.bfloat16)
    wqb = Wq.astype(jnp.bfloat16)
    kb = K_ext[0].reshape(SQ, D).astype(jnp.bfloat16)
    vb = V_ext[0].reshape(SQ, D).astype(jnp.bfloat16)
    wob = Wo.astype(jnp.bfloat16)

    def body(x_ref, wq_ref, k_ref, v_ref, wo_ref, out_ref,
             o_ref, l_ref, ctx_ref, ob_cw, ob_cc, lb_cw, lb_cc,
             rs_send_cw, rs_send_cc, rs_recv_cw, rs_recv_cc,
             rsl_send_cw, rsl_send_cc, rsl_recv_cw, rsl_recv_cc,
             ag_send_cw, ag_send_cc, ag_recv_cw, ag_recv_cc):
        me = lax.axis_index("i")
        left = (me - 1) % N_DEV
        right = (me + 1) % N_DEV

        barrier_sem = pltpu.get_barrier_semaphore()
        for nbr in (left, right):
            pl.semaphore_signal(barrier_sem, inc=1, device_id=(nbr,),
                                device_id_type=pl.DeviceIdType.MESH)
        pl.semaphore_wait(barrier_sem, 2)

        q = jnp.dot(x_ref[...], wq_ref[...],
                    preferred_element_type=jnp.float32)
        q16 = q.astype(jnp.bfloat16)

        k = k_ref[...]
        v = v_ref[...]
        for c in range(N_PHASE):
            qc = q16.reshape(8, 4, 64, D)[:, c].reshape(PH, D)
            kc = k.reshape(8, 4, 64, D)[:, c].reshape(PH, D)
            vc = v.reshape(8, 4, 64, D)[:, c].reshape(PH, D)
            for h in range(HQ):
                s = lax.dot_general(
                    qc[:, h * DH:(h + 1) * DH], kc[:, h * DH:(h + 1) * DH],
                    (((1,), (1,)), ((), ())),
                    preferred_element_type=jnp.float32)
                w = jnp.exp(s * SCALE)
                lsum = jnp.sum(w, axis=1)
                o = lax.dot_general(
                    w.astype(jnp.bfloat16), vc[:, h * DH:(h + 1) * DH],
                    (((1,), (0,)), ((), ())),
                    preferred_element_type=jnp.float32)
                o_ref[pl.ds(PH * c, PH), pl.ds(h * DH, DH)] = o
                l_ref[pl.ds(h, 1), pl.ds(PH * c, PH)] = lsum[None, :]

        prev_rs = None
        for t in range(N_DEV - 1):
            sc_cw = (me - 1 - t) % N_DEV
            rc_cw = (me - 2 - t) % N_DEV
            sc_cc = (me + 1 + t) % N_DEV
            rc_cc = (me + 2 + t) % N_DEV
            rdma_o_cw = pltpu.make_async_remote_copy(
                src_ref=o_ref.at[pl.ds(sc_cw * CHUNK, CHUNK), pl.ds(0, HALF)],
                dst_ref=ob_cw.at[t],
                send_sem=rs_send_cw, recv_sem=rs_recv_cw.at[t],
                device_id=(right,), device_id_type=pl.DeviceIdType.MESH)
            rdma_o_cc = pltpu.make_async_remote_copy(
                src_ref=o_ref.at[pl.ds(sc_cc * CHUNK, CHUNK),
                                 pl.ds(HALF, HALF)],
                dst_ref=ob_cc.at[t],
                send_sem=rs_send_cc, recv_sem=rs_recv_cc.at[t],
                device_id=(left,), device_id_type=pl.DeviceIdType.MESH)
            rdma_l_cw = pltpu.make_async_remote_copy(
                src_ref=l_ref.at[pl.ds(0, LH), pl.ds(sc_cw * CHUNK, CHUNK)],
                dst_ref=lb_cw.at[t],
                send_sem=rsl_send_cw, recv_sem=rsl_recv_cw.at[t],
                device_id=(right,), device_id_type=pl.DeviceIdType.MESH)
            rdma_l_cc = pltpu.make_async_remote_copy(
                src_ref=l_ref.at[pl.ds(LH, LH), pl.ds(sc_cc * CHUNK, CHUNK)],
                dst_ref=lb_cc.at[t],
                send_sem=rsl_send_cc, recv_sem=rsl_recv_cc.at[t],
                device_id=(left,), device_id_type=pl.DeviceIdType.MESH)
            if prev_rs is not None:
                for r in prev_rs:
                    r.wait_send()
            rdma_o_cw.start()
            rdma_o_cc.start()
            rdma_l_cw.start()
            rdma_l_cc.start()
            prev_rs = (rdma_o_cw, rdma_o_cc, rdma_l_cw, rdma_l_cc)
            rdma_o_cw.wait_recv()
            rdma_o_cc.wait_recv()
            rdma_l_cw.wait_recv()
            rdma_l_cc.wait_recv()
            o_ref[pl.ds(rc_cw * CHUNK, CHUNK), pl.ds(0, HALF)] = (
                ob_cw[t] + o_ref[pl.ds(rc_cw * CHUNK, CHUNK), pl.ds(0, HALF)])
            o_ref[pl.ds(rc_cc * CHUNK, CHUNK), pl.ds(HALF, HALF)] = (
                ob_cc[t]
                + o_ref[pl.ds(rc_cc * CHUNK, CHUNK), pl.ds(HALF, HALF)])
            l_ref[pl.ds(0, LH), pl.ds(rc_cw * CHUNK, CHUNK)] = (
                lb_cw[t] + l_ref[pl.ds(0, LH), pl.ds(rc_cw * CHUNK, CHUNK)])
            l_ref[pl.ds(LH, LH), pl.ds(rc_cc * CHUNK, CHUNK)] = (
                lb_cc[t] + l_ref[pl.ds(LH, LH), pl.ds(rc_cc * CHUNK, CHUNK)])
        for r in prev_rs:
            r.wait_send()

        oc = o_ref[pl.ds(me * CHUNK, CHUNK), :]
        lc = l_ref[:, pl.ds(me * CHUNK, CHUNK)]
        ctx = (oc.reshape(CHUNK, HQ, DH) / lc.T[:, :, None]).reshape(CHUNK, D)
        ctx_ref[pl.ds(me * CHUNK, CHUNK), :] = ctx.astype(jnp.bfloat16)

        prev_ag = None
        for t in range(N_DEV - 1):
            sc_cw = (me - t) % N_DEV
            sc_cc = (me + t) % N_DEV
            ag_cw = pltpu.make_async_remote_copy(
                src_ref=ctx_ref.at[pl.ds(sc_cw * CHUNK, CHUNK),
                                   pl.ds(0, HALF)],
                dst_ref=ctx_ref.at[pl.ds(sc_cw * CHUNK, CHUNK),
                                   pl.ds(0, HALF)],
                send_sem=ag_send_cw, recv_sem=ag_recv_cw.at[t],
                device_id=(right,), device_id_type=pl.DeviceIdType.MESH)
            ag_cc = pltpu.make_async_remote_copy(
                src_ref=ctx_ref.at[pl.ds(sc_cc * CHUNK, CHUNK),
                                   pl.ds(HALF, HALF)],
                dst_ref=ctx_ref.at[pl.ds(sc_cc * CHUNK, CHUNK),
                                   pl.ds(HALF, HALF)],
                send_sem=ag_send_cc, recv_sem=ag_recv_cc.at[t],
                device_id=(left,), device_id_type=pl.DeviceIdType.MESH)
            if prev_ag is not None:
                for r in prev_ag:
                    r.wait_send()
            ag_cw.start()
            ag_cc.start()
            prev_ag = (ag_cw, ag_cc)
            ag_cw.wait_recv()
            ag_cc.wait_recv()
        for r in prev_ag:
            r.wait_send()

        ctx_all = ctx_ref[...]
        ctx_nat = (ctx_all.reshape(N_PHASE, 8, 64, D)
                   .transpose(1, 0, 2, 3).reshape(SQ, D))
        out_ref[...] = jnp.dot(ctx_nat, wo_ref[...],
                               preferred_element_type=jnp.float32)

    out = pl.pallas_call(
        body,
        out_shape=jax.ShapeDtypeStruct((SQ, D), jnp.float32),
        in_specs=[pl.BlockSpec(memory_space=pltpu.VMEM)] * 5,
        out_specs=pl.BlockSpec(memory_space=pltpu.VMEM),
        scratch_shapes=[
            pltpu.VMEM((SQ, D), jnp.float32),
            pltpu.VMEM((HQ, SQ), jnp.float32),
            pltpu.VMEM((SQ, D), jnp.bfloat16),
            pltpu.VMEM((N_DEV - 1, CHUNK, HALF), jnp.float32),
            pltpu.VMEM((N_DEV - 1, CHUNK, HALF), jnp.float32),
            pltpu.VMEM((N_DEV - 1, LH, CHUNK), jnp.float32),
            pltpu.VMEM((N_DEV - 1, LH, CHUNK), jnp.float32),
            pltpu.SemaphoreType.DMA,
            pltpu.SemaphoreType.DMA,
            pltpu.SemaphoreType.DMA((N_DEV - 1,)),
            pltpu.SemaphoreType.DMA((N_DEV - 1,)),
            pltpu.SemaphoreType.DMA,
            pltpu.SemaphoreType.DMA,
            pltpu.SemaphoreType.DMA((N_DEV - 1,)),
            pltpu.SemaphoreType.DMA((N_DEV - 1,)),
            pltpu.SemaphoreType.DMA,
            pltpu.SemaphoreType.DMA,
            pltpu.SemaphoreType.DMA((N_DEV - 1,)),
            pltpu.SemaphoreType.DMA((N_DEV - 1,)),
        ],
        compiler_params=pltpu.CompilerParams(
            collective_id=0, vmem_limit_bytes=100 * 1024 * 1024),
    )(xb, wqb, kb, vb, wob)
    return out[None]


# device time: 144795 ns/iter; 1.7271x vs baseline; 1.5677x over previous
import jax
import jax.numpy as jnp
from jax import lax
from jax.experimental import pallas as pl
from jax.experimental.pallas import tpu as pltpu

N_DEV = 16
SQ = 2048
D = 1024
HQ = 8
DH = 128
N_PHASE = 4
PH = SQ // N_PHASE
CHUNK = SQ // N_DEV
PC = 136
SCALE = 0.08838834764831843

_PERM = [4 * i + c for c in range(4) for i in range(8)]


def kernel(x, Wq, K_ext, V_ext, Wo):
    idx = jnp.array(_PERM, dtype=jnp.int32)
    xb = x[0].reshape(32, 64, D)[idx].reshape(SQ, D).astype(jnp.bfloat16)
    wqb = Wq.astype(jnp.bfloat16)
    kb = (K_ext[0].reshape(32, 64, D)[idx].reshape(SQ, D)
          .astype(jnp.bfloat16))
    vb = (V_ext[0].reshape(32, 64, D)[idx].reshape(SQ, D)
          .astype(jnp.bfloat16))
    wob = Wo.astype(jnp.bfloat16)

    def body(x_ref, wq_ref, k_ref, v_ref, wo_ref, out_ref,
             po_ref, og_ref, pb_ref,
             rs_send, rs_recv, ag_send, ag_recv):
        me = lax.axis_index("i")

        barrier_sem = pltpu.get_barrier_semaphore()
        for d in range(N_DEV):
            @pl.when(me != d)
            def _():
                pl.semaphore_signal(barrier_sem, inc=1, device_id=(d,),
                                    device_id_type=pl.DeviceIdType.MESH)
        pl.semaphore_wait(barrier_sem, N_DEV - 1)

        rs_tx = {}
        for r in range(N_DEV):
            rs_tx[r] = pltpu.make_async_remote_copy(
                src_ref=po_ref.at[pl.ds(PC * r, PC), :],
                dst_ref=pb_ref.at[me],
                send_sem=rs_send.at[r], recv_sem=rs_recv.at[me],
                device_id=(r,), device_id_type=pl.DeviceIdType.MESH)

        q = jnp.dot(x_ref[...], wq_ref[...],
                    preferred_element_type=jnp.float32)
        q16 = q.astype(jnp.bfloat16)

        k = k_ref[...]
        v = v_ref[...]
        for c in range(N_PHASE):
            qc = q16[PH * c:PH * (c + 1)]
            kc = k[PH * c:PH * (c + 1)]
            vc = v[PH * c:PH * (c + 1)]
            for h in range(HQ):
                s = lax.dot_general(
                    qc[:, h * DH:(h + 1) * DH], kc[:, h * DH:(h + 1) * DH],
                    (((1,), (1,)), ((), ())),
                    preferred_element_type=jnp.float32)
                w = jnp.exp(s * SCALE)
                lsum = jnp.sum(w, axis=1)
                o = lax.dot_general(
                    w.astype(jnp.bfloat16), vc[:, h * DH:(h + 1) * DH],
                    (((1,), (0,)), ((), ())),
                    preferred_element_type=jnp.float32)
                o16 = o.astype(jnp.bfloat16)
                l16 = lsum.astype(jnp.bfloat16)
                for j in range(4):
                    r = 4 * c + j
                    po_ref[pl.ds(PC * r, CHUNK),
                           pl.ds(DH * h, DH)] = o16[CHUNK * j:CHUNK * (j + 1)]
                    po_ref[pl.ds(PC * r + CHUNK, 1), pl.ds(DH * h, DH)] = (
                        l16[CHUNK * j:CHUNK * (j + 1)][None, :])
            for j in range(4):
                r = 4 * c + j

                @pl.when(me != r)
                def _(r=r):
                    rs_tx[r].start()

        pb_ref[pl.ds(me, 1)] = po_ref[pl.ds(PC * me, PC), :][None]

        for s in range(N_DEV):
            @pl.when(me != s)
            def _(s=s):
                pltpu.make_async_remote_copy(
                    src_ref=pb_ref.at[s], dst_ref=pb_ref.at[s],
                    send_sem=rs_send.at[s], recv_sem=rs_recv.at[s],
                    device_id=(s,),
                    device_id_type=pl.DeviceIdType.MESH).wait_recv()
        acc = pb_ref[0].astype(jnp.float32)
        for s_ in range(1, N_DEV):
            acc = acc + pb_ref[s_].astype(jnp.float32)

        oc = acc[0:CHUNK]
        lc = acc[CHUNK].reshape(HQ, DH)
        ctx = (oc.reshape(CHUNK, HQ, DH) / lc.T[:, :, None]).reshape(CHUNK, D)
        out_mine = jnp.dot(ctx.astype(jnp.bfloat16), wo_ref[...],
                           preferred_element_type=jnp.float32)
        og_ref[pl.ds(CHUNK * me, CHUNK), :] = out_mine.astype(jnp.bfloat16)

        g1 = 8 * (me % 4) + me // 4
        out_ref[pl.ds(64 * g1, 64), :] = out_mine[0:64]
        out_ref[pl.ds(64 * (g1 + 4), 64), :] = out_mine[64:128]

        ag_tx = {}
        for d in range(N_DEV):
            ag_tx[d] = pltpu.make_async_remote_copy(
                src_ref=og_ref.at[pl.ds(CHUNK * me, CHUNK), :],
                dst_ref=og_ref.at[pl.ds(CHUNK * me, CHUNK), :],
                send_sem=ag_send.at[d], recv_sem=ag_recv.at[me],
                device_id=(d,), device_id_type=pl.DeviceIdType.MESH)

            @pl.when(me != d)
            def _(d=d):
                ag_tx[d].start()

        for s in range(N_DEV):
            @pl.when(me != s)
            def _(s=s):
                pltpu.make_async_remote_copy(
                    src_ref=og_ref.at[pl.ds(CHUNK * s, CHUNK), :],
                    dst_ref=og_ref.at[pl.ds(CHUNK * s, CHUNK), :],
                    send_sem=ag_send.at[s], recv_sem=ag_recv.at[s],
                    device_id=(s,),
                    device_id_type=pl.DeviceIdType.MESH).wait_recv()
                g = 8 * (s % 4) + s // 4
                out_ref[pl.ds(64 * g, 64), :] = (
                    og_ref[pl.ds(CHUNK * s, 64), :].astype(jnp.float32))
                out_ref[pl.ds(64 * (g + 4), 64), :] = (
                    og_ref[pl.ds(CHUNK * s + 64, 64), :].astype(jnp.float32))

        for r in range(N_DEV):
            @pl.when(me != r)
            def _(r=r):
                rs_tx[r].wait_send()
                ag_tx[r].wait_send()

    out = pl.pallas_call(
        body,
        out_shape=jax.ShapeDtypeStruct((SQ, D), jnp.float32),
        in_specs=[pl.BlockSpec(memory_space=pltpu.VMEM)] * 5,
        out_specs=pl.BlockSpec(memory_space=pltpu.VMEM),
        scratch_shapes=[
            pltpu.VMEM((N_DEV * PC, D), jnp.bfloat16),
            pltpu.VMEM((SQ, D), jnp.bfloat16),
            pltpu.VMEM((N_DEV, PC, D), jnp.bfloat16),
            pltpu.SemaphoreType.DMA((N_DEV,)),
            pltpu.SemaphoreType.DMA((N_DEV,)),
            pltpu.SemaphoreType.DMA((N_DEV,)),
            pltpu.SemaphoreType.DMA((N_DEV,)),
        ],
        compiler_params=pltpu.CompilerParams(
            collective_id=0, vmem_limit_bytes=100 * 1024 * 1024),
    )(xb, wqb, kb, vb, wob)
    return out[None]
